# baseline (speedup 1.0000x reference)
import jax
import jax.numpy as jnp
from jax import lax
from jax.experimental import pallas as pl
from jax.experimental.pallas import tpu as pltpu
from jax.experimental.pallas import tpu_sc as plsc

NUM_CLASSES = 1000
ROWS = 16384

_info = plsc.get_sparse_core_info()
NC, NS, L = _info.num_cores, _info.num_subcores, _info.num_lanes
NW = NC * NS
ROWS_PER_W = ROWS // NW
CHUNK = 64
NCHUNK = ROWS_PER_W // CHUNK
NBUF = 2


def _zero_buf(buf):
    zv = jnp.zeros((L,), jnp.float32)

    def body(i, carry):
        off = i * L
        for r in range(CHUNK):
            buf[r, pl.ds(off, L)] = zv
        return carry

    lax.fori_loop(0, NUM_CLASSES // L, body, 0)
    for r in range(CHUNK):
        buf[r, pl.ds(NUM_CLASSES - L, L)] = zv


def _scatter(buf, idx_v, chunk, val_vec):
    rows16 = lax.iota(jnp.int32, L)
    for k in range(CHUNK // L):
        cols = idx_v[pl.ds(chunk * CHUNK + k * L, L)]
        rloc = rows16 + (k * L)
        plsc.store_scatter(buf, [rloc, cols], val_vec)


def _onehot_sc(x_hbm, out_hbm, idx_v, buf0, buf1, sem0, sem1):
    wid = lax.axis_index("s") * NC + lax.axis_index("c")
    rowbase = wid * ROWS_PER_W
    pltpu.sync_copy(x_hbm.at[pl.ds(rowbase, ROWS_PER_W)], idx_v)

    ones = jnp.full((L,), 1.0, jnp.float32)
    zeros = jnp.zeros((L,), jnp.float32)
    bufs = (buf0, buf1)
    sems = (sem0, sem1)
    handles = [None] * NBUF

    _zero_buf(buf0)
    for c in range(NCHUNK):
        b = c % NBUF
        buf = bufs[b]
        if c == 1:
            _zero_buf(buf1)
        if handles[b] is not None:
            handles[b].wait()
            _scatter(buf, idx_v, c - NBUF, zeros)
        _scatter(buf, idx_v, c, ones)
        handles[b] = pltpu.async_copy(
            buf, out_hbm.at[pl.ds(rowbase + c * CHUNK, CHUNK)], sems[b]
        )
    for b in range(NBUF):
        handles[b].wait()


def kernel(x):
    xf = jnp.reshape(x, (ROWS,))
    mesh = plsc.VectorSubcoreMesh(core_axis_name="c", subcore_axis_name="s")
    out = pl.kernel(
        _onehot_sc,
        mesh=mesh,
        compiler_params=pltpu.CompilerParams(
            use_tc_tiling_on_sc=False, needs_layout_passes=False
        ),
        out_type=jax.ShapeDtypeStruct((ROWS, NUM_CLASSES), jnp.float32),
        scratch_types=[
            pltpu.VMEM((ROWS_PER_W,), jnp.int32),
            pltpu.VMEM((CHUNK, NUM_CLASSES), jnp.float32),
            pltpu.VMEM((CHUNK, NUM_CLASSES), jnp.float32),
            pltpu.SemaphoreType.DMA,
            pltpu.SemaphoreType.DMA,
        ],
    )(xf)
    return out


# speedup vs baseline: 1.6767x; 1.6767x over previous
import jax
import jax.numpy as jnp
from jax import lax
from jax.experimental import pallas as pl
from jax.experimental.pallas import tpu as pltpu
from jax.experimental.pallas import tpu_sc as plsc

NUM_CLASSES = 1000
ROWS = 16384

_info = plsc.get_sparse_core_info()
NC, NS, L = _info.num_cores, _info.num_subcores, _info.num_lanes
NW = NC * NS
ROWS_PER_W = ROWS // NW
CHUNK = 32
NCHUNK = ROWS_PER_W // CHUNK
NBUF = 2


def _zero_buf(buf):
    zv = jnp.zeros((L,), jnp.float32)

    def body(i, carry):
        off = i * L
        for r in range(CHUNK):
            buf[r, pl.ds(off, L)] = zv
        return carry

    lax.fori_loop(0, NUM_CLASSES // L, body, 0)
    for r in range(CHUNK):
        buf[r, pl.ds(NUM_CLASSES - L, L)] = zv


def _scatter(buf, idx_v, chunk, val_vec):
    rows16 = lax.iota(jnp.int32, L)
    for k in range(CHUNK // L):
        cols = idx_v[pl.ds(chunk * CHUNK + k * L, L)]
        rloc = rows16 + (k * L)
        plsc.store_scatter(buf, [rloc, cols], val_vec)


def _onehot_sc(x_hbm, out_hbm, idx_v, buf0, buf1, sem0, sem1):
    wid = lax.axis_index("s") * NC + lax.axis_index("c")
    rowbase = wid * ROWS_PER_W
    pltpu.sync_copy(x_hbm.at[pl.ds(rowbase, ROWS_PER_W)], idx_v)

    ones = jnp.full((L,), 1.0, jnp.float32)
    zeros = jnp.zeros((L,), jnp.float32)
    bufs = (buf0, buf1)
    sems = (sem0, sem1)
    handles = [None] * NBUF

    _zero_buf(buf0)
    for c in range(NCHUNK):
        b = c % NBUF
        buf = bufs[b]
        if c == 1:
            _zero_buf(buf1)
        if handles[b] is not None:
            handles[b].wait()
            _scatter(buf, idx_v, c - NBUF, zeros)
        _scatter(buf, idx_v, c, ones)
        handles[b] = pltpu.async_copy(
            buf, out_hbm.at[pl.ds(rowbase + c * CHUNK, CHUNK)], sems[b]
        )
    for b in range(NBUF):
        handles[b].wait()


def kernel(x):
    xf = jnp.reshape(x, (ROWS,))
    mesh = plsc.VectorSubcoreMesh(core_axis_name="c", subcore_axis_name="s")
    out = pl.kernel(
        _onehot_sc,
        mesh=mesh,
        compiler_params=pltpu.CompilerParams(
            use_tc_tiling_on_sc=True, needs_layout_passes=False
        ),
        out_type=jax.ShapeDtypeStruct((ROWS, NUM_CLASSES), jnp.float32),
        scratch_types=[
            pltpu.VMEM((ROWS_PER_W,), jnp.int32),
            pltpu.VMEM((CHUNK, NUM_CLASSES), jnp.float32),
            pltpu.VMEM((CHUNK, NUM_CLASSES), jnp.float32),
            pltpu.SemaphoreType.DMA,
            pltpu.SemaphoreType.DMA,
        ],
    )(xf)
    return out


# speedup vs baseline: 3.8814x; 2.3149x over previous
import jax
import jax.numpy as jnp
from jax import lax
from jax.experimental import pallas as pl
from jax.experimental.pallas import tpu as pltpu
from jax.experimental.pallas import tpu_sc as plsc

NUM_CLASSES = 1000
ROWS = 16384

_info = plsc.get_sparse_core_info()
NC, NS, L = _info.num_cores, _info.num_subcores, _info.num_lanes
NW = NC * NS
IPW = ROWS // NW
CC = 40
NCHUNK = NUM_CLASSES // CC
NBUF = 2
NGRP = IPW // L


def _zero_buf(buf):
    zv = jnp.zeros((L,), jnp.float32)

    def body(i, carry):
        off = i * L
        for r in range(CC):
            buf[r, pl.ds(off, L)] = zv
        return carry

    lax.fori_loop(0, IPW // L, body, 0)


def _onehot_sc(x_hbm, out_hbm, idx_v, buf0, buf1, sem0, sem1):
    wid = lax.axis_index("s") * NC + lax.axis_index("c")
    ibase = wid * IPW
    pltpu.sync_copy(x_hbm.at[pl.ds(ibase, IPW)], idx_v)

    iota = lax.iota(jnp.int32, L)
    ones = jnp.full((L,), 1.0, jnp.float32)
    zeros = jnp.zeros((L,), jnp.float32)
    bufs = (buf0, buf1)
    sems = (sem0, sem1)
    handles = [None] * NBUF

    for k in range(NCHUNK):
        b = k % NBUF
        buf = bufs[b]
        if k < NBUF:
            _zero_buf(buf)
        else:
            handles[b].wait()
        c_new = k * CC
        c_old = (k - NBUF) * CC

        def scan_body(g, carry, buf=buf, c_new=c_new, c_old=c_old, first=(k < NBUF)):
            v = idx_v[pl.ds(g * L, L)]
            cols = iota + g * L
            if not first:
                m_old = (v >= c_old) & (v < c_old + CC)
                plsc.store_scatter(buf, [v - c_old, cols], zeros, mask=m_old)
            m_new = (v >= c_new) & (v < c_new + CC)
            plsc.store_scatter(buf, [v - c_new, cols], ones, mask=m_new)
            return carry

        lax.fori_loop(0, NGRP, scan_body, 0)
        handles[b] = pltpu.async_copy(
            buf,
            out_hbm.at[pl.ds(c_new, CC), pl.ds(ibase, IPW)],
            sems[b],
        )
    for b in range(NBUF):
        handles[b].wait()


def kernel(x):
    xf = jnp.reshape(x, (ROWS,))
    mesh = plsc.VectorSubcoreMesh(core_axis_name="c", subcore_axis_name="s")
    out_t = pl.kernel(
        _onehot_sc,
        mesh=mesh,
        compiler_params=pltpu.CompilerParams(
            use_tc_tiling_on_sc=True, needs_layout_passes=False
        ),
        out_type=jax.ShapeDtypeStruct((NUM_CLASSES, ROWS), jnp.float32),
        scratch_types=[
            pltpu.VMEM((IPW,), jnp.int32),
            pltpu.VMEM((CC, IPW), jnp.float32),
            pltpu.VMEM((CC, IPW), jnp.float32),
            pltpu.SemaphoreType.DMA,
            pltpu.SemaphoreType.DMA,
        ],
    )(xf)
    return out_t.T


# speedup vs baseline: 3.9622x; 1.0208x over previous
import jax
import jax.numpy as jnp
from jax import lax
from jax.experimental import pallas as pl
from jax.experimental.pallas import tpu as pltpu
from jax.experimental.pallas import tpu_sc as plsc

NUM_CLASSES = 1000
ROWS = 16384

_info = plsc.get_sparse_core_info()
NC, NS, L = _info.num_cores, _info.num_subcores, _info.num_lanes
NW = NC * NS
IPW = ROWS // NW
CC = 40
NCHUNK = NUM_CLASSES // CC
NGRP = IPW // L


def _zero_buf(buf):
    zv = jnp.zeros((L,), jnp.float32)

    def body(i, carry):
        off = i * L
        for r in range(CC):
            buf[r, pl.ds(off, L)] = zv
        return carry

    lax.fori_loop(0, IPW // L, body, 0)


def _onehot_sc(x_hbm, out_hbm, idx_v, buf0, buf1, sem0, sem1):
    wid = lax.axis_index("s") * NC + lax.axis_index("c")
    ibase = wid * IPW
    pltpu.sync_copy(x_hbm.at[pl.ds(ibase, IPW)], idx_v)

    iota = lax.iota(jnp.int32, L)
    ones = jnp.full((L,), 1.0, jnp.float32)
    zeros = jnp.zeros((L,), jnp.float32)

    def scan(buf, c_new, c_old):
        def body(g, carry):
            v = idx_v[pl.ds(g * L, L)]
            cols = iota + g * L
            if c_old is not None:
                m_old = (v >= c_old) & (v < c_old + CC)
                plsc.store_scatter(buf, [v - c_old, cols], zeros, mask=m_old)
            m_new = (v >= c_new) & (v < c_new + CC)
            plsc.store_scatter(buf, [v - c_new, cols], ones, mask=m_new)
            return carry

        lax.fori_loop(0, NGRP, body, 0)

    def fire(buf, c_new, sem):
        return pltpu.async_copy(
            buf, out_hbm.at[pl.ds(c_new, CC), pl.ds(ibase, IPW)], sem
        )

    def drain(buf, c_new, sem):
        pltpu.make_async_copy(
            buf, out_hbm.at[pl.ds(c_new, CC), pl.ds(ibase, IPW)], sem
        ).wait()

    _zero_buf(buf0)
    scan(buf0, 0, None)
    fire(buf0, 0, sem0)
    _zero_buf(buf1)
    scan(buf1, CC, None)
    fire(buf1, CC, sem1)

    def pair(p, carry):
        k0 = 2 * p * CC
        drain(buf0, k0 - 2 * CC, sem0)
        scan(buf0, k0, k0 - 2 * CC)
        fire(buf0, k0, sem0)
        k1 = k0 + CC
        drain(buf1, k1 - 2 * CC, sem1)
        scan(buf1, k1, k1 - 2 * CC)
        fire(buf1, k1, sem1)
        return carry

    lax.fori_loop(1, (NCHUNK - 1) // 2, pair, 0)

    c_last = (NCHUNK - 1) * CC
    drain(buf0, c_last - 2 * CC, sem0)
    scan(buf0, c_last, c_last - 2 * CC)
    fire(buf0, c_last, sem0)
    drain(buf1, (NCHUNK - 2) * CC, sem1)
    drain(buf0, c_last, sem0)


def kernel(x):
    xf = jnp.reshape(x, (ROWS,))
    mesh = plsc.VectorSubcoreMesh(core_axis_name="c", subcore_axis_name="s")
    out_t = pl.kernel(
        _onehot_sc,
        mesh=mesh,
        compiler_params=pltpu.CompilerParams(
            use_tc_tiling_on_sc=True, needs_layout_passes=False
        ),
        out_type=jax.ShapeDtypeStruct((NUM_CLASSES, ROWS), jnp.float32),
        scratch_types=[
            pltpu.VMEM((IPW,), jnp.int32),
            pltpu.VMEM((CC, IPW), jnp.float32),
            pltpu.VMEM((CC, IPW), jnp.float32),
            pltpu.SemaphoreType.DMA,
            pltpu.SemaphoreType.DMA,
        ],
    )(xf)
    return out_t.T


# speedup vs baseline: 3.9788x; 1.0042x over previous
import jax
import jax.numpy as jnp
from jax import lax
from jax.experimental import pallas as pl
from jax.experimental.pallas import tpu as pltpu
from jax.experimental.pallas import tpu_sc as plsc

NUM_CLASSES = 1000
ROWS = 16384

_info = plsc.get_sparse_core_info()
NC, NS, L = _info.num_cores, _info.num_subcores, _info.num_lanes
NW = NC * NS
IPW = ROWS // NW
CC = 40
NCHUNK = NUM_CLASSES // CC
NBUF = 3
NGRP = IPW // L


def _zero_buf(buf):
    zv = jnp.zeros((L,), jnp.float32)

    def body(i, carry):
        off = i * L
        for r in range(CC):
            buf[r, pl.ds(off, L)] = zv
        return carry

    lax.fori_loop(0, IPW // L, body, 0)


def _onehot_sc(x_hbm, out_hbm, idx_v, buf0, buf1, buf2, sem0, sem1, sem2, isem):
    wid = lax.axis_index("s") * NC + lax.axis_index("c")
    ibase = wid * IPW
    idx_dma = pltpu.async_copy(x_hbm.at[pl.ds(ibase, IPW)], idx_v, isem)

    iota = lax.iota(jnp.int32, L)
    ones = jnp.full((L,), 1.0, jnp.float32)
    zeros = jnp.zeros((L,), jnp.float32)
    bufs = (buf0, buf1, buf2)
    sems = (sem0, sem1, sem2)
    handles = [None] * NBUF

    for k in range(NCHUNK):
        b = k % NBUF
        buf = bufs[b]
        if k < NBUF:
            _zero_buf(buf)
            if k == 0:
                idx_dma.wait()
        else:
            handles[b].wait()
        c_new = k * CC
        c_old = (k - NBUF) * CC

        def scan_body(g, carry, buf=buf, c_new=c_new, c_old=c_old, first=(k < NBUF)):
            v = idx_v[pl.ds(g * L, L)]
            cols = iota + g * L
            if not first:
                m_old = (v >= c_old) & (v < c_old + CC)
                plsc.store_scatter(buf, [v - c_old, cols], zeros, mask=m_old)
            m_new = (v >= c_new) & (v < c_new + CC)
            plsc.store_scatter(buf, [v - c_new, cols], ones, mask=m_new)
            return carry

        lax.fori_loop(0, NGRP, scan_body, 0)
        handles[b] = pltpu.async_copy(
            buf,
            out_hbm.at[pl.ds(c_new, CC), pl.ds(ibase, IPW)],
            sems[b],
        )
    for b in range(NBUF):
        handles[b].wait()


def kernel(x):
    xf = jnp.reshape(x, (ROWS,))
    mesh = plsc.VectorSubcoreMesh(core_axis_name="c", subcore_axis_name="s")
    out_t = pl.kernel(
        _onehot_sc,
        mesh=mesh,
        compiler_params=pltpu.CompilerParams(
            use_tc_tiling_on_sc=True, needs_layout_passes=False
        ),
        out_type=jax.ShapeDtypeStruct((NUM_CLASSES, ROWS), jnp.float32),
        scratch_types=[
            pltpu.VMEM((IPW,), jnp.int32),
            pltpu.VMEM((CC, IPW), jnp.float32),
            pltpu.VMEM((CC, IPW), jnp.float32),
            pltpu.VMEM((CC, IPW), jnp.float32),
            pltpu.SemaphoreType.DMA,
            pltpu.SemaphoreType.DMA,
            pltpu.SemaphoreType.DMA,
            pltpu.SemaphoreType.DMA,
        ],
    )(xf)
    return out_t.T


# speedup vs baseline: 7.5022x; 1.8855x over previous
import jax
import jax.numpy as jnp
from jax.experimental import pallas as pl

NUM_CLASSES = 1000
ROWS = 16384
BLOCK_COLS = 2048


def _onehot_block(x_ref, o_ref):
    idx = x_ref[0]
    rows = jax.lax.broadcasted_iota(jnp.int32, (NUM_CLASSES, BLOCK_COLS), 0)
    o_ref[...] = (rows == idx).astype(jnp.float32)


def kernel(x):
    grid = ROWS // BLOCK_COLS
    xf = jnp.reshape(x, (grid, 1, BLOCK_COLS))
    out_t = pl.pallas_call(
        _onehot_block,
        grid=(grid,),
        in_specs=[pl.BlockSpec((1, 1, BLOCK_COLS), lambda i: (i, 0, 0))],
        out_specs=pl.BlockSpec((NUM_CLASSES, BLOCK_COLS), lambda i: (0, i)),
        out_shape=jax.ShapeDtypeStruct((NUM_CLASSES, ROWS), jnp.float32),
    )(xf)
    return out_t.T
